# baseline (speedup 1.0000x reference)
import jax
import jax.numpy as jnp
from jax.experimental import pallas as pl
from jax.experimental.pallas import tpu as pltpu

E = 320000
D = 128
BLOCK = 2560


def _encode_block(a_ref, b_ref, w1_ref, w2_ref, o_ref):
    acc = jnp.dot(a_ref[:], w1_ref[:], preferred_element_type=jnp.float32)
    acc = acc + jnp.dot(b_ref[:], w2_ref[:], preferred_element_type=jnp.float32)
    o_ref[:] = jnp.maximum(acc, 0.0)


def kernel(edges_attr, edges_nb_attr, W):
    w1 = W[:D, :]
    w2 = W[D:, :]
    grid = (E // BLOCK,)
    return pl.pallas_call(
        _encode_block,
        grid=grid,
        in_specs=[
            pl.BlockSpec((BLOCK, D), lambda i: (i, 0)),
            pl.BlockSpec((BLOCK, D), lambda i: (i, 0)),
            pl.BlockSpec((D, D), lambda i: (0, 0)),
            pl.BlockSpec((D, D), lambda i: (0, 0)),
        ],
        out_specs=pl.BlockSpec((BLOCK, D), lambda i: (i, 0)),
        out_shape=jax.ShapeDtypeStruct((E, D), jnp.float32),
        compiler_params=pltpu.CompilerParams(
            dimension_semantics=("arbitrary",),
        ),
    )(edges_attr, edges_nb_attr, w1, w2)


# speedup vs baseline: 1.1658x; 1.1658x over previous
import jax
import jax.numpy as jnp
from jax.experimental import pallas as pl
from jax.experimental.pallas import tpu as pltpu

E = 320000
D = 128
BLOCK = 4000


def _encode_block(a_ref, b_ref, w1_ref, w2_ref, o_ref):
    acc = jnp.dot(a_ref[:], w1_ref[:], preferred_element_type=jnp.float32)
    acc = acc + jnp.dot(b_ref[:], w2_ref[:], preferred_element_type=jnp.float32)
    o_ref[:] = jnp.maximum(acc, 0.0)


def kernel(edges_attr, edges_nb_attr, W):
    w1 = W[:D, :]
    w2 = W[D:, :]
    grid = (E // BLOCK,)
    return pl.pallas_call(
        _encode_block,
        grid=grid,
        in_specs=[
            pl.BlockSpec((BLOCK, D), lambda i: (i, 0)),
            pl.BlockSpec((BLOCK, D), lambda i: (i, 0)),
            pl.BlockSpec((D, D), lambda i: (0, 0)),
            pl.BlockSpec((D, D), lambda i: (0, 0)),
        ],
        out_specs=pl.BlockSpec((BLOCK, D), lambda i: (i, 0)),
        out_shape=jax.ShapeDtypeStruct((E, D), jnp.float32),
        compiler_params=pltpu.CompilerParams(
            dimension_semantics=("parallel",),
        ),
    )(edges_attr, edges_nb_attr, w1, w2)


# speedup vs baseline: 1.2317x; 1.0565x over previous
import jax
import jax.numpy as jnp
from jax.experimental import pallas as pl
from jax.experimental.pallas import tpu as pltpu

E = 320000
D = 128
BLOCK = 8000


def _encode_block(a_ref, b_ref, w1_ref, w2_ref, o_ref):
    acc = jnp.dot(a_ref[:], w1_ref[:], preferred_element_type=jnp.float32)
    acc = acc + jnp.dot(b_ref[:], w2_ref[:], preferred_element_type=jnp.float32)
    o_ref[:] = jnp.maximum(acc, 0.0)


def kernel(edges_attr, edges_nb_attr, W):
    w1 = W[:D, :]
    w2 = W[D:, :]
    grid = (E // BLOCK,)
    return pl.pallas_call(
        _encode_block,
        grid=grid,
        in_specs=[
            pl.BlockSpec((BLOCK, D), lambda i: (i, 0)),
            pl.BlockSpec((BLOCK, D), lambda i: (i, 0)),
            pl.BlockSpec((D, D), lambda i: (0, 0)),
            pl.BlockSpec((D, D), lambda i: (0, 0)),
        ],
        out_specs=pl.BlockSpec((BLOCK, D), lambda i: (i, 0)),
        out_shape=jax.ShapeDtypeStruct((E, D), jnp.float32),
        compiler_params=pltpu.CompilerParams(
            dimension_semantics=("parallel",),
        ),
    )(edges_attr, edges_nb_attr, w1, w2)


# speedup vs baseline: 1.2400x; 1.0067x over previous
import jax
import jax.numpy as jnp
from jax.experimental import pallas as pl
from jax.experimental.pallas import tpu as pltpu

E = 320000
D = 128
BLOCK = 16000


def _encode_block(a_ref, b_ref, w1_ref, w2_ref, o_ref):
    acc = jnp.dot(a_ref[:], w1_ref[:], preferred_element_type=jnp.float32)
    acc = acc + jnp.dot(b_ref[:], w2_ref[:], preferred_element_type=jnp.float32)
    o_ref[:] = jnp.maximum(acc, 0.0)


def kernel(edges_attr, edges_nb_attr, W):
    w1 = W[:D, :]
    w2 = W[D:, :]
    grid = (E // BLOCK,)
    return pl.pallas_call(
        _encode_block,
        grid=grid,
        in_specs=[
            pl.BlockSpec((BLOCK, D), lambda i: (i, 0)),
            pl.BlockSpec((BLOCK, D), lambda i: (i, 0)),
            pl.BlockSpec((D, D), lambda i: (0, 0)),
            pl.BlockSpec((D, D), lambda i: (0, 0)),
        ],
        out_specs=pl.BlockSpec((BLOCK, D), lambda i: (i, 0)),
        out_shape=jax.ShapeDtypeStruct((E, D), jnp.float32),
        compiler_params=pltpu.CompilerParams(
            dimension_semantics=("parallel",),
        ),
    )(edges_attr, edges_nb_attr, w1, w2)


# speedup vs baseline: 1.2563x; 1.0132x over previous
import jax
import jax.numpy as jnp
from jax.experimental import pallas as pl
from jax.experimental.pallas import tpu as pltpu

E = 320000
D = 128
BLOCK = 16000


def _encode_block(a_ref, b_ref, w_ref, o_ref):
    acc = jnp.dot(a_ref[:], w_ref[0:D, :], preferred_element_type=jnp.float32)
    acc = acc + jnp.dot(b_ref[:], w_ref[D:, :], preferred_element_type=jnp.float32)
    o_ref[:] = jnp.maximum(acc, 0.0)


def kernel(edges_attr, edges_nb_attr, W):
    grid = (E // BLOCK,)
    return pl.pallas_call(
        _encode_block,
        grid=grid,
        in_specs=[
            pl.BlockSpec((BLOCK, D), lambda i: (i, 0)),
            pl.BlockSpec((BLOCK, D), lambda i: (i, 0)),
            pl.BlockSpec((2 * D, D), lambda i: (0, 0)),
        ],
        out_specs=pl.BlockSpec((BLOCK, D), lambda i: (i, 0)),
        out_shape=jax.ShapeDtypeStruct((E, D), jnp.float32),
        compiler_params=pltpu.CompilerParams(
            dimension_semantics=("parallel",),
        ),
    )(edges_attr, edges_nb_attr, W)


# speedup vs baseline: 1.2597x; 1.0027x over previous
import jax
import jax.numpy as jnp
from jax.experimental import pallas as pl
from jax.experimental.pallas import tpu as pltpu

E = 320000
D = 128
BLOCK = 10000


def _encode_block(a_ref, b_ref, w_ref, o_ref):
    acc = jnp.dot(a_ref[:], w_ref[0:D, :], preferred_element_type=jnp.float32)
    acc = acc + jnp.dot(b_ref[:], w_ref[D:, :], preferred_element_type=jnp.float32)
    o_ref[:] = jnp.maximum(acc, 0.0)


def kernel(edges_attr, edges_nb_attr, W):
    grid = (E // BLOCK,)
    return pl.pallas_call(
        _encode_block,
        grid=grid,
        in_specs=[
            pl.BlockSpec((BLOCK, D), lambda i: (i, 0)),
            pl.BlockSpec((BLOCK, D), lambda i: (i, 0)),
            pl.BlockSpec((2 * D, D), lambda i: (0, 0)),
        ],
        out_specs=pl.BlockSpec((BLOCK, D), lambda i: (i, 0)),
        out_shape=jax.ShapeDtypeStruct((E, D), jnp.float32),
        compiler_params=pltpu.CompilerParams(
            dimension_semantics=("parallel",),
        ),
    )(edges_attr, edges_nb_attr, W)
